# Initial kernel scaffold; baseline (speedup 1.0000x reference)
import jax
import jax.numpy as jnp
from jax.experimental import pallas as pl


def kernel(x, edge_index, W_l1, b_l1, W_r1, W_l2, b_l2, W_r2, fc_W, fc_b):
    raise NotImplementedError("write your pallas kernel here")


# speedup vs baseline: 7.1375x; 7.1375x over previous
import functools

import jax
import jax.numpy as jnp
from jax import lax
from jax.experimental import pallas as pl
from jax.experimental.pallas import tpu as pltpu
from jax.experimental.pallas import tpu_sc as plsc

_N = 10000
_E = 320000
_D = 128
_HH = 2048
_AUG = 144
_NSC = 2
_NTEC = 16
_NW = _NSC * _NTEC
_CHUNK = 128
_NCHUNK = 79
_EPAD = _NW * _NCHUNK * _CHUNK
_RPT = 632
_NACC = _RPT * _NTEC
_BLK = 1000


def _seg_body(table, srcp, dstp, zrows, out, src_v, dst_v, buf, acc, sem):
    c = lax.axis_index("c")
    s = lax.axis_index("s")
    w = c * _NTEC + s
    pltpu.sync_copy(zrows, acc.at[pl.ds(s * _RPT, _RPT)])
    pltpu.sync_copy(srcp.at[w], src_v)
    pltpu.sync_copy(dstp.at[w], dst_v)
    plsc.subcore_barrier()

    def body(j, carry):
        pltpu.async_copy(table.at[src_v.at[j]], buf, sem).wait()
        pltpu.sync_copy(buf, acc.at[dst_v.at[j]], add=True)
        return carry

    lax.fori_loop(0, _NCHUNK, body, 0)
    plsc.subcore_barrier()
    pltpu.sync_copy(acc.at[pl.ds(s * _RPT, _RPT)],
                    out.at[c, pl.ds(s * _RPT, _RPT)])


@functools.cache
def _seg():
    return pl.kernel(
        _seg_body,
        out_type=jax.ShapeDtypeStruct((_NSC, _NACC, _AUG), jnp.float32),
        mesh=plsc.VectorSubcoreMesh(core_axis_name="c", subcore_axis_name="s"),
        compiler_params=pltpu.CompilerParams(use_tc_tiling_on_sc=False),
        scratch_types=[
            pltpu.VMEM((_NCHUNK, _CHUNK), jnp.int32),
            pltpu.VMEM((_NCHUNK, _CHUNK), jnp.int32),
            pltpu.VMEM((_CHUNK, _AUG), jnp.float32),
            pltpu.VMEM_SHARED((_NACC, _AUG), jnp.float32),
            pltpu.SemaphoreType.DMA,
        ],
    )


def _ones_col(rows):
    col = lax.broadcasted_iota(jnp.int32, (rows, _AUG - _D), 1)
    return jnp.where(col == 0, 1.0, 0.0).astype(jnp.float32)


def _k1_body(x_ref, wl_ref, wr_ref, bl_ref, t_ref, r_ref):
    xb = x_ref[...]
    t = jnp.dot(xb, wl_ref[...], preferred_element_type=jnp.float32)
    t_ref[...] = jnp.concatenate([t, _ones_col(xb.shape[0])], axis=1)
    r_ref[...] = (jnp.dot(xb, wr_ref[...], preferred_element_type=jnp.float32)
                  + bl_ref[...])


def _agg_h(p_ref, r_ref):
    p = p_ref[0] + p_ref[1]
    cnt = jnp.clip(p[:, _D:_D + 1], 1.0, None)
    return jnp.maximum(p[:, :_D] / cnt + r_ref[...], 0.0)


def _k2_body(p_ref, r_ref, wl_ref, wr_ref, bl_ref, t_ref, r2_ref):
    h = _agg_h(p_ref, r_ref)
    t = jnp.dot(h, wl_ref[...], preferred_element_type=jnp.float32)
    t_ref[...] = jnp.concatenate([t, _ones_col(h.shape[0])], axis=1)
    r2_ref[...] = (jnp.dot(h, wr_ref[...], preferred_element_type=jnp.float32)
                   + bl_ref[...])


def _k3_body(p_ref, r_ref, fw_ref, fb_ref, o_ref):
    h = _agg_h(p_ref, r_ref)
    o_ref[...] = (jnp.dot(h, fw_ref[...], preferred_element_type=jnp.float32)
                  + fb_ref[...])


_GRID = _N // _BLK

_w_spec = pl.BlockSpec((_D, _D), lambda i: (0, 0))
_b_spec = pl.BlockSpec((1, _D), lambda i: (0, 0))
_row_spec = pl.BlockSpec((_BLK, _D), lambda i: (i, 0))
_aug_spec = pl.BlockSpec((_BLK, _AUG), lambda i: (i, 0))
_p_spec = pl.BlockSpec((_NSC, _BLK, _AUG), lambda i: (0, i, 0))

_k1 = pl.pallas_call(
    _k1_body,
    grid=(_GRID,),
    in_specs=[_row_spec, _w_spec, _w_spec, _b_spec],
    out_specs=[_aug_spec, _row_spec],
    out_shape=[jax.ShapeDtypeStruct((_N, _AUG), jnp.float32),
               jax.ShapeDtypeStruct((_N, _D), jnp.float32)],
)

_k2 = pl.pallas_call(
    _k2_body,
    grid=(_GRID,),
    in_specs=[_p_spec, _row_spec, _w_spec, _w_spec, _b_spec],
    out_specs=[_aug_spec, _row_spec],
    out_shape=[jax.ShapeDtypeStruct((_N, _AUG), jnp.float32),
               jax.ShapeDtypeStruct((_N, _D), jnp.float32)],
)

_k3 = pl.pallas_call(
    _k3_body,
    grid=(_GRID,),
    in_specs=[_p_spec, _row_spec,
              pl.BlockSpec((_D, _HH), lambda i: (0, 0)),
              pl.BlockSpec((1, _HH), lambda i: (0, 0))],
    out_specs=pl.BlockSpec((_BLK, _HH), lambda i: (i, 0)),
    out_shape=jax.ShapeDtypeStruct((_N, _HH), jnp.float32),
)


def kernel(x, edge_index, W_l1, b_l1, W_r1, W_l2, b_l2, W_r2, fc_W, fc_b):
    src = edge_index[0]
    dst = edge_index[1]
    npad = _EPAD - _E
    ar = jnp.arange(npad, dtype=jnp.int32)
    pad_src = (ar * 37) % _N
    pad_dst = _N + (ar % (_NACC - _N))
    srcp = jnp.concatenate([src, pad_src]).reshape(_NW, _NCHUNK, _CHUNK)
    dstp = jnp.concatenate([dst, pad_dst]).reshape(_NW, _NCHUNK, _CHUNK)
    zrows = jnp.zeros((_RPT, _AUG), jnp.float32)

    seg = _seg()
    t1, r1 = _k1(x, W_l1, W_r1, b_l1.reshape(1, _D))
    p1 = seg(t1, srcp, dstp, zrows)
    t2, r2 = _k2(p1, r1, W_l2, W_r2, b_l2.reshape(1, _D))
    p2 = seg(t2, srcp, dstp, zrows)
    return _k3(p2, r2, fc_W, fc_b.reshape(1, _HH))


# speedup vs baseline: 8.2042x; 1.1495x over previous
import functools

import jax
import jax.numpy as jnp
from jax import lax
from jax.experimental import pallas as pl
from jax.experimental.pallas import tpu as pltpu
from jax.experimental.pallas import tpu_sc as plsc

_N = 10000
_E = 320000
_D = 128
_HH = 2048
_AUG = 144
_NSC = 2
_NTEC = 16
_NW = _NSC * _NTEC
_CHUNK = 128
_NCHUNK = 80
_EPAD = _NW * _NCHUNK * _CHUNK
_RPT = 632
_NACC = _RPT * _NTEC
_BLK = 1000


def _seg_body(table, sdp, zrows, out, idx0, idx1, buf0, buf1, acc,
              sg0, sg1, si0, si1):
    c = lax.axis_index("c")
    s = lax.axis_index("s")
    w = c * _NTEC + s
    pltpu.sync_copy(zrows, acc.at[pl.ds(s * _RPT, _RPT)])
    plsc.subcore_barrier()

    pltpu.sync_copy(sdp.at[w, 0], idx0)
    pltpu.async_copy(table.at[idx0.at[0]], buf0, sg0)
    pltpu.async_copy(sdp.at[w, 1], idx1, si1)

    def body(i, carry):
        j = 2 * i
        pltpu.make_async_copy(sdp.at[w, j + 1], idx1, si1).wait()
        pltpu.async_copy(table.at[idx1.at[0]], buf1, sg1)
        pltpu.make_async_copy(table.at[idx0.at[0]], buf0, sg0).wait()
        pltpu.sync_copy(buf0, acc.at[idx0.at[1]], add=True)

        @pl.when(j + 2 < _NCHUNK)
        def _():
            pltpu.async_copy(sdp.at[w, j + 2], idx0, si0)

        pltpu.make_async_copy(table.at[idx1.at[0]], buf1, sg1).wait()
        pltpu.sync_copy(buf1, acc.at[idx1.at[1]], add=True)

        @pl.when(j + 2 < _NCHUNK)
        def _():
            pltpu.make_async_copy(sdp.at[w, j + 2], idx0, si0).wait()
            pltpu.async_copy(table.at[idx0.at[0]], buf0, sg0)
            pltpu.async_copy(sdp.at[w, j + 3], idx1, si1)

        return carry

    lax.fori_loop(0, _NCHUNK // 2, body, 0)
    plsc.subcore_barrier()
    pltpu.sync_copy(acc.at[pl.ds(s * _RPT, _RPT)],
                    out.at[c, pl.ds(s * _RPT, _RPT)])


@functools.cache
def _seg():
    return pl.kernel(
        _seg_body,
        out_type=jax.ShapeDtypeStruct((_NSC, _NACC, _AUG), jnp.float32),
        mesh=plsc.VectorSubcoreMesh(core_axis_name="c", subcore_axis_name="s"),
        compiler_params=pltpu.CompilerParams(use_tc_tiling_on_sc=False),
        scratch_types=[
            pltpu.VMEM((2, _CHUNK), jnp.int32),
            pltpu.VMEM((2, _CHUNK), jnp.int32),
            pltpu.VMEM((_CHUNK, _AUG), jnp.float32),
            pltpu.VMEM((_CHUNK, _AUG), jnp.float32),
            pltpu.VMEM_SHARED((_NACC, _AUG), jnp.float32),
            pltpu.SemaphoreType.DMA,
            pltpu.SemaphoreType.DMA,
            pltpu.SemaphoreType.DMA,
            pltpu.SemaphoreType.DMA,
        ],
    )


def _ones_col(rows):
    col = lax.broadcasted_iota(jnp.int32, (rows, _AUG - _D), 1)
    return jnp.where(col == 0, 1.0, 0.0).astype(jnp.float32)


def _k1_body(x_ref, wl_ref, wr_ref, bl_ref, t_ref, r_ref):
    xb = x_ref[...]
    t = jnp.dot(xb, wl_ref[...], preferred_element_type=jnp.float32)
    t_ref[...] = jnp.concatenate([t, _ones_col(xb.shape[0])], axis=1)
    r_ref[...] = (jnp.dot(xb, wr_ref[...], preferred_element_type=jnp.float32)
                  + bl_ref[...])


def _agg_h(p_ref, r_ref):
    p = p_ref[0] + p_ref[1]
    cnt = jnp.clip(p[:, _D:_D + 1], 1.0, None)
    return jnp.maximum(p[:, :_D] / cnt + r_ref[...], 0.0)


def _k2_body(p_ref, r_ref, wl_ref, wr_ref, bl_ref, t_ref, r2_ref):
    h = _agg_h(p_ref, r_ref)
    t = jnp.dot(h, wl_ref[...], preferred_element_type=jnp.float32)
    t_ref[...] = jnp.concatenate([t, _ones_col(h.shape[0])], axis=1)
    r2_ref[...] = (jnp.dot(h, wr_ref[...], preferred_element_type=jnp.float32)
                   + bl_ref[...])


def _k3_body(p_ref, r_ref, fw_ref, fb_ref, o_ref):
    h = _agg_h(p_ref, r_ref)
    o_ref[...] = (jnp.dot(h, fw_ref[...], preferred_element_type=jnp.float32)
                  + fb_ref[...])


_GRID = _N // _BLK

_w_spec = pl.BlockSpec((_D, _D), lambda i: (0, 0))
_b_spec = pl.BlockSpec((1, _D), lambda i: (0, 0))
_row_spec = pl.BlockSpec((_BLK, _D), lambda i: (i, 0))
_aug_spec = pl.BlockSpec((_BLK, _AUG), lambda i: (i, 0))
_p_spec = pl.BlockSpec((_NSC, _BLK, _AUG), lambda i: (0, i, 0))

_k1 = pl.pallas_call(
    _k1_body,
    grid=(_GRID,),
    in_specs=[_row_spec, _w_spec, _w_spec, _b_spec],
    out_specs=[_aug_spec, _row_spec],
    out_shape=[jax.ShapeDtypeStruct((_N, _AUG), jnp.float32),
               jax.ShapeDtypeStruct((_N, _D), jnp.float32)],
)

_k2 = pl.pallas_call(
    _k2_body,
    grid=(_GRID,),
    in_specs=[_p_spec, _row_spec, _w_spec, _w_spec, _b_spec],
    out_specs=[_aug_spec, _row_spec],
    out_shape=[jax.ShapeDtypeStruct((_N, _AUG), jnp.float32),
               jax.ShapeDtypeStruct((_N, _D), jnp.float32)],
)

_k3 = pl.pallas_call(
    _k3_body,
    grid=(_GRID,),
    in_specs=[_p_spec, _row_spec,
              pl.BlockSpec((_D, _HH), lambda i: (0, 0)),
              pl.BlockSpec((1, _HH), lambda i: (0, 0))],
    out_specs=pl.BlockSpec((_BLK, _HH), lambda i: (i, 0)),
    out_shape=jax.ShapeDtypeStruct((_N, _HH), jnp.float32),
)


def kernel(x, edge_index, W_l1, b_l1, W_r1, W_l2, b_l2, W_r2, fc_W, fc_b):
    src = edge_index[0]
    dst = edge_index[1]
    npad = _EPAD - _E
    ar = jnp.arange(npad, dtype=jnp.int32)
    pad_src = (ar * 37) % _N
    pad_dst = _N + (ar % (_NACC - _N))
    srcp = jnp.concatenate([src, pad_src]).reshape(_NW, _NCHUNK, _CHUNK)
    dstp = jnp.concatenate([dst, pad_dst]).reshape(_NW, _NCHUNK, _CHUNK)
    sdp = jnp.stack([srcp, dstp], axis=2)
    zrows = jnp.zeros((_RPT, _AUG), jnp.float32)

    seg = _seg()
    t1, r1 = _k1(x, W_l1, W_r1, b_l1.reshape(1, _D))
    p1 = seg(t1, sdp, zrows)
    t2, r2 = _k2(p1, r1, W_l2, W_r2, b_l2.reshape(1, _D))
    p2 = seg(t2, sdp, zrows)
    return _k3(p2, r2, fc_W, fc_b.reshape(1, _HH))


# speedup vs baseline: 8.3388x; 1.0164x over previous
import functools

import jax
import jax.numpy as jnp
from jax import lax
from jax.experimental import pallas as pl
from jax.experimental.pallas import tpu as pltpu
from jax.experimental.pallas import tpu_sc as plsc

_N = 10000
_E = 320000
_D = 128
_HH = 2048
_AUG = 144
_NSC = 2
_NTEC = 16
_NW = _NSC * _NTEC
_CHUNK = 128
_NCHUNK = 80
_EPAD = _NW * _NCHUNK * _CHUNK
_RPT = 632
_NACC = _RPT * _NTEC
_BLK = 1000


def _seg_body(table, sdp, zrows, out, idxa, idxb, buf0, buf1, acc,
              sga, sgb, ss, sia, sib):
    c = lax.axis_index("c")
    s = lax.axis_index("s")
    w = c * _NTEC + s
    pltpu.sync_copy(zrows, acc.at[pl.ds(s * _RPT, _RPT)])
    pltpu.async_copy(sdp.at[w, 0, 0], idxa, sia)
    plsc.subcore_barrier()

    def gat(q, idx, buf, sem):
        pltpu.async_copy(table.at[idx.at[2 * q]], buf, sem)

    def gat_w(q, idx, buf, sem):
        pltpu.make_async_copy(table.at[idx.at[2 * q]], buf, sem).wait()

    def sca(q, idx, buf, sem):
        pltpu.async_copy(buf, acc.at[idx.at[2 * q + 1]], sem, add=True)

    def sca_w(q, idx, buf, sem):
        pltpu.make_async_copy(buf, acc.at[idx.at[2 * q + 1]], sem).wait()

    def body(i, carry):
        pltpu.make_async_copy(sdp.at[w, i, 0], idxa, sia).wait()

        @pl.when(i > 0)
        def _():
            sca_w(3, idxb, buf1, ss)

        gat(0, idxa, buf0, sga)
        gat(1, idxa, buf1, sgb)
        pltpu.async_copy(sdp.at[w, i, 1], idxb, sib)
        gat_w(0, idxa, buf0, sga)
        sca(0, idxa, buf0, ss)
        gat_w(1, idxa, buf1, sgb)
        sca_w(0, idxa, buf0, ss)
        sca(1, idxa, buf1, ss)
        gat(2, idxa, buf0, sga)
        gat_w(2, idxa, buf0, sga)
        sca_w(1, idxa, buf1, ss)
        sca(2, idxa, buf0, ss)
        gat(3, idxa, buf1, sgb)
        pltpu.make_async_copy(sdp.at[w, i, 1], idxb, sib).wait()
        gat_w(3, idxa, buf1, sgb)
        sca_w(2, idxa, buf0, ss)
        sca(3, idxa, buf1, ss)
        gat(0, idxb, buf0, sga)
        gat_w(0, idxb, buf0, sga)
        sca_w(3, idxa, buf1, ss)
        sca(0, idxb, buf0, ss)
        gat(1, idxb, buf1, sgb)

        @pl.when(i + 1 < _NCHUNK // 8)
        def _():
            pltpu.async_copy(sdp.at[w, i + 1, 0], idxa, sia)

        gat_w(1, idxb, buf1, sgb)
        sca_w(0, idxb, buf0, ss)
        sca(1, idxb, buf1, ss)
        gat(2, idxb, buf0, sga)
        gat_w(2, idxb, buf0, sga)
        sca_w(1, idxb, buf1, ss)
        sca(2, idxb, buf0, ss)
        gat(3, idxb, buf1, sgb)
        gat_w(3, idxb, buf1, sgb)
        sca_w(2, idxb, buf0, ss)
        sca(3, idxb, buf1, ss)
        return carry

    lax.fori_loop(0, _NCHUNK // 8, body, 0)
    sca_w(3, idxb, buf1, ss)
    plsc.subcore_barrier()
    pltpu.sync_copy(acc.at[pl.ds(s * _RPT, _RPT)],
                    out.at[c, pl.ds(s * _RPT, _RPT)])


@functools.cache
def _seg():
    return pl.kernel(
        _seg_body,
        out_type=jax.ShapeDtypeStruct((_NSC, _NACC, _AUG), jnp.float32),
        mesh=plsc.VectorSubcoreMesh(core_axis_name="c", subcore_axis_name="s"),
        compiler_params=pltpu.CompilerParams(use_tc_tiling_on_sc=False),
        scratch_types=[
            pltpu.VMEM((8, _CHUNK), jnp.int32),
            pltpu.VMEM((8, _CHUNK), jnp.int32),
            pltpu.VMEM((_CHUNK, _AUG), jnp.float32),
            pltpu.VMEM((_CHUNK, _AUG), jnp.float32),
            pltpu.VMEM_SHARED((_NACC, _AUG), jnp.float32),
            pltpu.SemaphoreType.DMA,
            pltpu.SemaphoreType.DMA,
            pltpu.SemaphoreType.DMA,
            pltpu.SemaphoreType.DMA,
            pltpu.SemaphoreType.DMA,
        ],
    )


def _ones_col(rows):
    col = lax.broadcasted_iota(jnp.int32, (rows, _AUG - _D), 1)
    return jnp.where(col == 0, 1.0, 0.0).astype(jnp.float32)


def _k1_body(x_ref, wl_ref, wr_ref, bl_ref, t_ref, r_ref):
    xb = x_ref[...]
    t = jnp.dot(xb, wl_ref[...], preferred_element_type=jnp.float32)
    t_ref[...] = jnp.concatenate([t, _ones_col(xb.shape[0])], axis=1)
    r_ref[...] = (jnp.dot(xb, wr_ref[...], preferred_element_type=jnp.float32)
                  + bl_ref[...])


def _agg_h(p_ref, r_ref):
    p = p_ref[0] + p_ref[1]
    cnt = jnp.clip(p[:, _D:_D + 1], 1.0, None)
    return jnp.maximum(p[:, :_D] / cnt + r_ref[...], 0.0)


def _k2_body(p_ref, r_ref, wl_ref, wr_ref, bl_ref, t_ref, r2_ref):
    h = _agg_h(p_ref, r_ref)
    t = jnp.dot(h, wl_ref[...], preferred_element_type=jnp.float32)
    t_ref[...] = jnp.concatenate([t, _ones_col(h.shape[0])], axis=1)
    r2_ref[...] = (jnp.dot(h, wr_ref[...], preferred_element_type=jnp.float32)
                   + bl_ref[...])


def _k3_body(p_ref, r_ref, fw_ref, fb_ref, o_ref):
    h = _agg_h(p_ref, r_ref)
    o_ref[...] = (jnp.dot(h, fw_ref[...], preferred_element_type=jnp.float32)
                  + fb_ref[...])


_GRID = _N // _BLK

_w_spec = pl.BlockSpec((_D, _D), lambda i: (0, 0))
_b_spec = pl.BlockSpec((1, _D), lambda i: (0, 0))
_row_spec = pl.BlockSpec((_BLK, _D), lambda i: (i, 0))
_aug_spec = pl.BlockSpec((_BLK, _AUG), lambda i: (i, 0))
_p_spec = pl.BlockSpec((_NSC, _BLK, _AUG), lambda i: (0, i, 0))

_k1 = pl.pallas_call(
    _k1_body,
    grid=(_GRID,),
    in_specs=[_row_spec, _w_spec, _w_spec, _b_spec],
    out_specs=[_aug_spec, _row_spec],
    out_shape=[jax.ShapeDtypeStruct((_N, _AUG), jnp.float32),
               jax.ShapeDtypeStruct((_N, _D), jnp.float32)],
)

_k2 = pl.pallas_call(
    _k2_body,
    grid=(_GRID,),
    in_specs=[_p_spec, _row_spec, _w_spec, _w_spec, _b_spec],
    out_specs=[_aug_spec, _row_spec],
    out_shape=[jax.ShapeDtypeStruct((_N, _AUG), jnp.float32),
               jax.ShapeDtypeStruct((_N, _D), jnp.float32)],
)

_k3 = pl.pallas_call(
    _k3_body,
    grid=(_GRID,),
    in_specs=[_p_spec, _row_spec,
              pl.BlockSpec((_D, _HH), lambda i: (0, 0)),
              pl.BlockSpec((1, _HH), lambda i: (0, 0))],
    out_specs=pl.BlockSpec((_BLK, _HH), lambda i: (i, 0)),
    out_shape=jax.ShapeDtypeStruct((_N, _HH), jnp.float32),
)


def kernel(x, edge_index, W_l1, b_l1, W_r1, W_l2, b_l2, W_r2, fc_W, fc_b):
    src = edge_index[0]
    dst = edge_index[1]
    npad = _EPAD - _E
    ar = jnp.arange(npad, dtype=jnp.int32)
    pad_src = (ar * 37) % _N
    pad_dst = _N + (ar % (_NACC - _N))
    srcp = jnp.concatenate([src, pad_src]).reshape(_NW, _NCHUNK, _CHUNK)
    dstp = jnp.concatenate([dst, pad_dst]).reshape(_NW, _NCHUNK, _CHUNK)
    sdp = jnp.stack([srcp, dstp], axis=2).reshape(
        _NW, _NCHUNK // 8, 2, 8, _CHUNK)
    zrows = jnp.zeros((_RPT, _AUG), jnp.float32)

    seg = _seg()
    t1, r1 = _k1(x, W_l1, W_r1, b_l1.reshape(1, _D))
    p1 = seg(t1, sdp, zrows)
    t2, r2 = _k2(p1, r1, W_l2, W_r2, b_l2.reshape(1, _D))
    p2 = seg(t2, sdp, zrows)
    return _k3(p2, r2, fc_W, fc_b.reshape(1, _HH))


# speedup vs baseline: 9.1729x; 1.1000x over previous
import functools

import jax
import jax.numpy as jnp
from jax import lax
from jax.experimental import pallas as pl
from jax.experimental.pallas import tpu as pltpu
from jax.experimental.pallas import tpu_sc as plsc

_N = 10000
_E = 320000
_D = 128
_HH = 2048
_AUG = 144
_NSC = 2
_NTEC = 16
_NW = _NSC * _NTEC
_CHUNK = 128
_NCHUNK = 80
_EPAD = _NW * _NCHUNK * _CHUNK
_RPT = 632
_NACC = _RPT * _NTEC
_BLK = 1000


def _make_seg_body(aug):
  def _seg_body(table, sdp, zrows, out, idxa, idxb, buf0, buf1, acc,
                  sga, sgb, ss, sia, sib):
      c = lax.axis_index("c")
      s = lax.axis_index("s")
      w = c * _NTEC + s
      pltpu.sync_copy(zrows, acc.at[pl.ds(s * _RPT, _RPT)])
      pltpu.async_copy(sdp.at[w, 0, 0], idxa, sia)
      plsc.subcore_barrier()

      def gat(q, idx, buf, sem):
          pltpu.async_copy(table.at[idx.at[2 * q]], buf, sem)

      def gat_w(q, idx, buf, sem):
          pltpu.make_async_copy(table.at[idx.at[2 * q]], buf, sem).wait()

      def sca(q, idx, buf, sem):
          pltpu.async_copy(buf, acc.at[idx.at[2 * q + 1]], sem, add=True)

      def sca_w(q, idx, buf, sem):
          pltpu.make_async_copy(buf, acc.at[idx.at[2 * q + 1]], sem).wait()

      def body(i, carry):
          pltpu.make_async_copy(sdp.at[w, i, 0], idxa, sia).wait()

          @pl.when(i > 0)
          def _():
              sca_w(3, idxb, buf1, ss)

          gat(0, idxa, buf0, sga)
          gat(1, idxa, buf1, sgb)
          pltpu.async_copy(sdp.at[w, i, 1], idxb, sib)
          gat_w(0, idxa, buf0, sga)
          sca(0, idxa, buf0, ss)
          gat_w(1, idxa, buf1, sgb)
          sca_w(0, idxa, buf0, ss)
          sca(1, idxa, buf1, ss)
          gat(2, idxa, buf0, sga)
          gat_w(2, idxa, buf0, sga)
          sca_w(1, idxa, buf1, ss)
          sca(2, idxa, buf0, ss)
          gat(3, idxa, buf1, sgb)
          pltpu.make_async_copy(sdp.at[w, i, 1], idxb, sib).wait()
          gat_w(3, idxa, buf1, sgb)
          sca_w(2, idxa, buf0, ss)
          sca(3, idxa, buf1, ss)
          gat(0, idxb, buf0, sga)
          gat_w(0, idxb, buf0, sga)
          sca_w(3, idxa, buf1, ss)
          sca(0, idxb, buf0, ss)
          gat(1, idxb, buf1, sgb)

          @pl.when(i + 1 < _NCHUNK // 8)
          def _():
              pltpu.async_copy(sdp.at[w, i + 1, 0], idxa, sia)

          gat_w(1, idxb, buf1, sgb)
          sca_w(0, idxb, buf0, ss)
          sca(1, idxb, buf1, ss)
          gat(2, idxb, buf0, sga)
          gat_w(2, idxb, buf0, sga)
          sca_w(1, idxb, buf1, ss)
          sca(2, idxb, buf0, ss)
          gat(3, idxb, buf1, sgb)
          gat_w(3, idxb, buf1, sgb)
          sca_w(2, idxb, buf0, ss)
          sca(3, idxb, buf1, ss)
          return carry

      lax.fori_loop(0, _NCHUNK // 8, body, 0)
      sca_w(3, idxb, buf1, ss)
      plsc.subcore_barrier()
      pltpu.sync_copy(acc.at[pl.ds(s * _RPT, _RPT)],
                      out.at[c, pl.ds(s * _RPT, _RPT)])

  return _seg_body


@functools.cache
def _seg(aug, tc_tiling):
    return pl.kernel(
        _make_seg_body(aug),
        out_type=jax.ShapeDtypeStruct((_NSC, _NACC, aug), jnp.float32),
        mesh=plsc.VectorSubcoreMesh(core_axis_name="c", subcore_axis_name="s"),
        compiler_params=pltpu.CompilerParams(use_tc_tiling_on_sc=tc_tiling),
        scratch_types=[
            pltpu.VMEM((8, _CHUNK), jnp.int32),
            pltpu.VMEM((8, _CHUNK), jnp.int32),
            pltpu.VMEM((_CHUNK, aug), jnp.float32),
            pltpu.VMEM((_CHUNK, aug), jnp.float32),
            pltpu.VMEM_SHARED((_NACC, aug), jnp.float32),
            pltpu.SemaphoreType.DMA,
            pltpu.SemaphoreType.DMA,
            pltpu.SemaphoreType.DMA,
            pltpu.SemaphoreType.DMA,
            pltpu.SemaphoreType.DMA,
        ],
    )


def _ones_col(rows):
    col = lax.broadcasted_iota(jnp.int32, (rows, _AUG - _D), 1)
    return jnp.where(col == 0, 1.0, 0.0).astype(jnp.float32)


def _k1_body(x_ref, wl_ref, wr_ref, bl_ref, t_ref, r_ref):
    xb = x_ref[...]
    t = jnp.dot(xb, wl_ref[...], preferred_element_type=jnp.float32)
    t_ref[...] = jnp.concatenate([t, _ones_col(xb.shape[0])], axis=1)
    r_ref[...] = (jnp.dot(xb, wr_ref[...], preferred_element_type=jnp.float32)
                  + bl_ref[...])


def _agg_h(p_ref, r_ref):
    p = p_ref[0] + p_ref[1]
    cnt = jnp.clip(p[:, _D:_D + 1], 1.0, None)
    return jnp.maximum(p[:, :_D] / cnt + r_ref[...], 0.0)


def _k2_body(p_ref, r_ref, wl_ref, wr_ref, bl_ref, t_ref, r2_ref):
    h = _agg_h(p_ref, r_ref)
    t_ref[...] = jnp.dot(h, wl_ref[...], preferred_element_type=jnp.float32)
    r2_ref[...] = (jnp.dot(h, wr_ref[...], preferred_element_type=jnp.float32)
                   + bl_ref[...])


def _k3_body(p_ref, p1_ref, r_ref, fw_ref, fb_ref, o_ref):
    p = p_ref[0] + p_ref[1]
    cnt = jnp.clip(p1_ref[0, :, _D:_D + 1] + p1_ref[1, :, _D:_D + 1], 1.0,
                   None)
    h = jnp.maximum(p / cnt + r_ref[...], 0.0)
    o_ref[...] = (jnp.dot(h, fw_ref[...], preferred_element_type=jnp.float32)
                  + fb_ref[...])


_GRID = _N // _BLK

_w_spec = pl.BlockSpec((_D, _D), lambda i: (0, 0))
_b_spec = pl.BlockSpec((1, _D), lambda i: (0, 0))
_row_spec = pl.BlockSpec((_BLK, _D), lambda i: (i, 0))
_aug_spec = pl.BlockSpec((_BLK, _AUG), lambda i: (i, 0))
_p_spec = pl.BlockSpec((_NSC, _BLK, _AUG), lambda i: (0, i, 0))
_p128_spec = pl.BlockSpec((_NSC, _BLK, _D), lambda i: (0, i, 0))

_k1 = pl.pallas_call(
    _k1_body,
    grid=(_GRID,),
    in_specs=[_row_spec, _w_spec, _w_spec, _b_spec],
    out_specs=[_aug_spec, _row_spec],
    out_shape=[jax.ShapeDtypeStruct((_N, _AUG), jnp.float32),
               jax.ShapeDtypeStruct((_N, _D), jnp.float32)],
)

_k2 = pl.pallas_call(
    _k2_body,
    grid=(_GRID,),
    in_specs=[_p_spec, _row_spec, _w_spec, _w_spec, _b_spec],
    out_specs=[_row_spec, _row_spec],
    out_shape=[jax.ShapeDtypeStruct((_N, _D), jnp.float32),
               jax.ShapeDtypeStruct((_N, _D), jnp.float32)],
)

_k3 = pl.pallas_call(
    _k3_body,
    grid=(_GRID,),
    in_specs=[_p128_spec, _p_spec, _row_spec,
              pl.BlockSpec((_D, _HH), lambda i: (0, 0)),
              pl.BlockSpec((1, _HH), lambda i: (0, 0))],
    out_specs=pl.BlockSpec((_BLK, _HH), lambda i: (i, 0)),
    out_shape=jax.ShapeDtypeStruct((_N, _HH), jnp.float32),
)


def kernel(x, edge_index, W_l1, b_l1, W_r1, W_l2, b_l2, W_r2, fc_W, fc_b):
    src = edge_index[0]
    dst = edge_index[1]
    npad = _EPAD - _E
    ar = jnp.arange(npad, dtype=jnp.int32)
    pad_src = (ar * 37) % _N
    pad_dst = _N + (ar % (_NACC - _N))
    srcp = jnp.concatenate([src, pad_src]).reshape(_NW, _NCHUNK, _CHUNK)
    dstp = jnp.concatenate([dst, pad_dst]).reshape(_NW, _NCHUNK, _CHUNK)
    sdp = jnp.stack([srcp, dstp], axis=2).reshape(
        _NW, _NCHUNK // 8, 2, 8, _CHUNK)
    t1, r1 = _k1(x, W_l1, W_r1, b_l1.reshape(1, _D))
    p1 = _seg(_AUG, False)(t1, sdp, jnp.zeros((_RPT, _AUG), jnp.float32))
    t2, r2 = _k2(p1, r1, W_l2, W_r2, b_l2.reshape(1, _D))
    p2 = _seg(_D, True)(t2, sdp, jnp.zeros((_RPT, _D), jnp.float32))
    return _k3(p2, p1, r2, fc_W, fc_b.reshape(1, _HH))
